# baseline (device time: 94653 ns/iter reference)
import jax
import jax.numpy as jnp
from jax import lax
from jax.experimental import pallas as pl
from jax.experimental.pallas import tpu as pltpu

B, SQ, H, D = 4, 32, 8, 128
SCALE = D ** -0.5
CHUNK = 1024
MESHID = pl.DeviceIdType.MESH


def kernel(Q, K, V):
    skv = K.shape[1]
    nc = skv // CHUNK

    def body(q_ref, k_ref, v_ref, out_ref,
             o_acc, l_acc, o_part, l_part, acc_o, acc_l,
             ro0, rl0, ro1, rl1,
             s0o, r0o, s0l, r0l, s1o, r1o, s1l, r1l):
        b = pl.program_id(0)
        c = pl.program_id(1)
        my_x = lax.axis_index("x")
        my_y = lax.axis_index("y")
        my_z = lax.axis_index("z")
        p1 = (my_x, my_y, my_z ^ 1)
        p2 = (my_x, my_y, my_z ^ 2)

        def r0(slot):
            return pltpu.make_async_remote_copy(
                src_ref=o_part.at[slot], dst_ref=ro0.at[slot],
                send_sem=s0o.at[slot], recv_sem=r0o.at[slot],
                device_id=p1, device_id_type=MESHID)

        def r0_l(slot):
            return pltpu.make_async_remote_copy(
                src_ref=l_part.at[slot], dst_ref=rl0.at[slot],
                send_sem=s0l.at[slot], recv_sem=r0l.at[slot],
                device_id=p1, device_id_type=MESHID)

        def r1(slot):
            return pltpu.make_async_remote_copy(
                src_ref=acc_o.at[slot], dst_ref=ro1.at[slot],
                send_sem=s1o.at[slot], recv_sem=r1o.at[slot],
                device_id=p2, device_id_type=MESHID)

        def r1_l(slot):
            return pltpu.make_async_remote_copy(
                src_ref=acc_l.at[slot], dst_ref=rl1.at[slot],
                send_sem=s1l.at[slot], recv_sem=r1l.at[slot],
                device_id=p2, device_id_type=MESHID)

        @pl.when(jnp.logical_and(b == 0, c == 0))
        def _():
            bar = pltpu.get_barrier_semaphore()
            for p in (p1, p2):
                pl.semaphore_signal(bar, inc=1, device_id=p,
                                    device_id_type=MESHID)
            pl.semaphore_wait(bar, 2)

        @pl.when(c == 0)
        def _():
            o_acc[...] = jnp.zeros_like(o_acc)
            l_acc[...] = jnp.zeros_like(l_acc)

        for h in range(H):
            q = q_ref[0, :, h, :]
            k = k_ref[0, :, h, :]
            v = v_ref[0, :, h, :]
            s = lax.dot_general(q, k, (((1,), (1,)), ((), ())),
                                preferred_element_type=jnp.float32) * SCALE
            p = jnp.exp(s)
            l_acc[:, h] += jnp.sum(p, axis=1)
            o_acc[:, h, :] += lax.dot_general(
                p, v, (((1,), (0,)), ((), ())),
                preferred_element_type=jnp.float32)

        @pl.when(c == nc - 1)
        def _():
            o_part[b] = o_acc[...]
            l_part[b] = l_acc[...]
            r0(b).start()
            r0_l(b).start()

        @pl.when(jnp.logical_and(b >= 1, c == 2))
        def _():
            bp = b - 1
            r0(bp).wait()
            r0_l(bp).wait()
            acc_o[bp] = o_part[bp] + ro0[bp]
            acc_l[bp] = l_part[bp] + rl0[bp]
            r1(bp).start()
            r1_l(bp).start()

        @pl.when(jnp.logical_and(b == B - 1, c == nc - 1))
        def _():
            last = B - 1
            r0(last).wait()
            r0_l(last).wait()
            acc_o[last] = o_part[last] + ro0[last]
            acc_l[last] = l_part[last] + rl0[last]
            r1(last).start()
            r1_l(last).start()
            for bb in range(B):
                r1(bb).wait()
                r1_l(bb).wait()
                num = acc_o[bb] + ro1[bb]
                den = (acc_l[bb] + rl1[bb])[..., None]
                out_ref[bb] = num / den

    return pl.pallas_call(
        body,
        grid=(B, nc),
        in_specs=[
            pl.BlockSpec((1, SQ, H, D), lambda b, c: (b, 0, 0, 0)),
            pl.BlockSpec((1, CHUNK, H, D), lambda b, c: (b, c, 0, 0)),
            pl.BlockSpec((1, CHUNK, H, D), lambda b, c: (b, c, 0, 0)),
        ],
        out_specs=pl.BlockSpec((B, SQ, H, D), lambda b, c: (0, 0, 0, 0)),
        out_shape=jax.ShapeDtypeStruct((B, SQ, H, D), jnp.float32),
        scratch_shapes=[
            pltpu.VMEM((SQ, H, D), jnp.float32),
            pltpu.VMEM((SQ, H), jnp.float32),
            pltpu.VMEM((B, SQ, H, D), jnp.float32),
            pltpu.VMEM((B, SQ, H), jnp.float32),
            pltpu.VMEM((B, SQ, H, D), jnp.float32),
            pltpu.VMEM((B, SQ, H), jnp.float32),
            pltpu.VMEM((B, SQ, H, D), jnp.float32),
            pltpu.VMEM((B, SQ, H), jnp.float32),
            pltpu.VMEM((B, SQ, H, D), jnp.float32),
            pltpu.VMEM((B, SQ, H), jnp.float32),
            pltpu.SemaphoreType.DMA((B,)),
            pltpu.SemaphoreType.DMA((B,)),
            pltpu.SemaphoreType.DMA((B,)),
            pltpu.SemaphoreType.DMA((B,)),
            pltpu.SemaphoreType.DMA((B,)),
            pltpu.SemaphoreType.DMA((B,)),
            pltpu.SemaphoreType.DMA((B,)),
            pltpu.SemaphoreType.DMA((B,)),
        ],
        compiler_params=pltpu.CompilerParams(collective_id=0),
    )(Q, K, V)


# device time: 74072 ns/iter; 1.2779x vs baseline; 1.2779x over previous
import jax
import jax.numpy as jnp
from jax import lax
from jax.experimental import pallas as pl
from jax.experimental.pallas import tpu as pltpu

B, SQ, H, D = 4, 32, 8, 128
SCALE = D ** -0.5
CHUNK = 512
N_ROUNDS = 5
MESHID = pl.DeviceIdType.MESH


def kernel(Q, K, V):
    rep = (lax.axis_index("x") * 4 + lax.axis_index("y")).astype(jnp.int32)
    rep = rep.reshape((1,))

    def body(rep_ref, q_ref, k_ref, v_ref, out_ref,
             acc_o, acc_l, ro, rl, so_s, so_r, sl_s, sl_r):
        b = pl.program_id(0)
        my_x = lax.axis_index("x")
        my_y = lax.axis_index("y")
        my_z = lax.axis_index("z")
        partners = [
            (my_x ^ 1, my_y, my_z),
            (my_x, my_y ^ 1, my_z),
            (my_x, my_y, my_z ^ 1),
            (my_x, my_y ^ 2, my_z),
            (my_x, my_y, my_z ^ 2),
        ]

        @pl.when(b == 0)
        def _():
            bar = pltpu.get_barrier_semaphore()
            for p in partners:
                pl.semaphore_signal(bar, inc=1, device_id=p,
                                    device_id_type=MESHID)
            pl.semaphore_wait(bar, N_ROUNDS)

        for h in range(H):
            q = q_ref[0, :, h, :]
            k = k_ref[0, :, h, :]
            v = v_ref[0, :, h, :]
            s = lax.dot_general(q, k, (((1,), (1,)), ((), ())),
                                preferred_element_type=jnp.float32) * SCALE
            p = jnp.exp(s)
            acc_l[b, :, h] = jnp.sum(p, axis=1)
            acc_o[b, :, h, :] = lax.dot_general(
                p, v, (((1,), (0,)), ((), ())),
                preferred_element_type=jnp.float32)

        @pl.when(b == B - 1)
        def _():
            for r, p in enumerate(partners):
                c_o = pltpu.make_async_remote_copy(
                    src_ref=acc_o, dst_ref=ro.at[r],
                    send_sem=so_s.at[r], recv_sem=so_r.at[r],
                    device_id=p, device_id_type=MESHID)
                c_l = pltpu.make_async_remote_copy(
                    src_ref=acc_l, dst_ref=rl.at[r],
                    send_sem=sl_s.at[r], recv_sem=sl_r.at[r],
                    device_id=p, device_id_type=MESHID)
                c_o.start()
                c_l.start()
                c_o.wait()
                c_l.wait()
                acc_o[...] += ro[r]
                acc_l[...] += rl[r]
            out_ref[...] = acc_o[...] / acc_l[...][..., None]

    grid_spec = pltpu.PrefetchScalarGridSpec(
        num_scalar_prefetch=1,
        grid=(B,),
        in_specs=[
            pl.BlockSpec((1, SQ, H, D), lambda b, rep_ref: (b, 0, 0, 0)),
            pl.BlockSpec((1, CHUNK, H, D),
                         lambda b, rep_ref: (b, rep_ref[0], 0, 0)),
            pl.BlockSpec((1, CHUNK, H, D),
                         lambda b, rep_ref: (b, rep_ref[0], 0, 0)),
        ],
        out_specs=pl.BlockSpec((B, SQ, H, D), lambda b, rep_ref: (0, 0, 0, 0)),
        scratch_shapes=[
            pltpu.VMEM((B, SQ, H, D), jnp.float32),
            pltpu.VMEM((B, SQ, H), jnp.float32),
            pltpu.VMEM((N_ROUNDS, B, SQ, H, D), jnp.float32),
            pltpu.VMEM((N_ROUNDS, B, SQ, H), jnp.float32),
            pltpu.SemaphoreType.DMA((N_ROUNDS,)),
            pltpu.SemaphoreType.DMA((N_ROUNDS,)),
            pltpu.SemaphoreType.DMA((N_ROUNDS,)),
            pltpu.SemaphoreType.DMA((N_ROUNDS,)),
        ],
    )

    return pl.pallas_call(
        body,
        grid_spec=grid_spec,
        out_shape=jax.ShapeDtypeStruct((B, SQ, H, D), jnp.float32),
        compiler_params=pltpu.CompilerParams(collective_id=0),
    )(rep, Q, K, V)


# device time: 52806 ns/iter; 1.7925x vs baseline; 1.4027x over previous
import jax
import jax.numpy as jnp
from jax import lax
from jax.experimental import pallas as pl
from jax.experimental.pallas import tpu as pltpu

B, SQ, H, D = 4, 32, 8, 128
ROWS = B * SQ * H
SCALE = D ** -0.5
CHUNK = 512
MESHID = pl.DeviceIdType.MESH
RS_SIZES = (512, 256, 128, 64, 32)


def kernel(Q, K, V):
    rep = (lax.axis_index("x") * 4 + lax.axis_index("y")).astype(jnp.int32)
    rep = rep.reshape((1,))

    def body(rep_ref, q_ref, k_ref, v_ref, out_ref,
             acc_o, acc_l, o_tmp, rbuf, rl, so_s, so_r, sl_s, sl_r):
        b = pl.program_id(0)
        my_x = lax.axis_index("x")
        my_y = lax.axis_index("y")
        my_z = lax.axis_index("z")
        partners = [
            (my_x ^ 1, my_y, my_z),
            (my_x, my_y ^ 1, my_z),
            (my_x, my_y, my_z ^ 1),
            (my_x, my_y ^ 2, my_z),
            (my_x, my_y, my_z ^ 2),
        ]
        bits = [
            my_x & 1,
            my_y & 1,
            my_z & 1,
            (my_y // 2) & 1,
            (my_z // 2) & 1,
        ]

        @pl.when(b == 0)
        def _():
            bar = pltpu.get_barrier_semaphore()
            for p in partners:
                pl.semaphore_signal(bar, inc=1, device_id=p,
                                    device_id_type=MESHID)
            pl.semaphore_wait(bar, len(partners))

        for h in range(H):
            q = q_ref[0, :, h, :]
            k = k_ref[0, :, h, :]
            v = v_ref[0, :, h, :]
            s = lax.dot_general(q, k, (((1,), (1,)), ((), ())),
                                preferred_element_type=jnp.float32) * SCALE
            p = jnp.exp(s)
            acc_l[b, :, h] = jnp.sum(p, axis=1)
            o_tmp[:, h, :] = lax.dot_general(
                p, v, (((1,), (0,)), ((), ())),
                preferred_element_type=jnp.float32)
        acc_o[pl.ds(b * SQ * H, SQ * H)] = o_tmp[...].reshape(SQ * H, D)

        @pl.when(b == B - 1)
        def _():
            cur_off = jnp.int32(0)
            for r in range(5):
                hr = RS_SIZES[r]
                br = bits[r]
                send_off = cur_off + (1 - br) * hr
                keep_off = cur_off + br * hr
                c_o = pltpu.make_async_remote_copy(
                    src_ref=acc_o.at[pl.ds(send_off, hr)],
                    dst_ref=rbuf.at[r, pl.ds(0, hr)],
                    send_sem=so_s.at[r], recv_sem=so_r.at[r],
                    device_id=partners[r], device_id_type=MESHID)
                c_l = pltpu.make_async_remote_copy(
                    src_ref=acc_l, dst_ref=rl.at[r],
                    send_sem=sl_s.at[r], recv_sem=sl_r.at[r],
                    device_id=partners[r], device_id_type=MESHID)
                c_o.start()
                c_l.start()
                c_o.wait()
                c_l.wait()
                acc_o[pl.ds(keep_off, hr)] += rbuf[r, 0:hr]
                acc_l[...] += rl[r]
                cur_off = keep_off
            for i in range(5):
                r = 5 + i
                d = 4 - i
                gr = RS_SIZES[d]
                c_o = pltpu.make_async_remote_copy(
                    src_ref=acc_o.at[pl.ds(cur_off, gr)],
                    dst_ref=acc_o.at[pl.ds(cur_off, gr)],
                    send_sem=so_s.at[r], recv_sem=so_r.at[r],
                    device_id=partners[d], device_id_type=MESHID)
                c_o.start()
                c_o.wait()
                cur_off = cur_off - bits[d] * gr
            num = acc_o[...].reshape(B, SQ, H, D)
            out_ref[...] = num / acc_l[...][..., None]

    grid_spec = pltpu.PrefetchScalarGridSpec(
        num_scalar_prefetch=1,
        grid=(B,),
        in_specs=[
            pl.BlockSpec((1, SQ, H, D), lambda b, rep_ref: (b, 0, 0, 0)),
            pl.BlockSpec((1, CHUNK, H, D),
                         lambda b, rep_ref: (b, rep_ref[0], 0, 0)),
            pl.BlockSpec((1, CHUNK, H, D),
                         lambda b, rep_ref: (b, rep_ref[0], 0, 0)),
        ],
        out_specs=pl.BlockSpec((B, SQ, H, D), lambda b, rep_ref: (0, 0, 0, 0)),
        scratch_shapes=[
            pltpu.VMEM((ROWS, D), jnp.float32),
            pltpu.VMEM((B, SQ, H), jnp.float32),
            pltpu.VMEM((SQ, H, D), jnp.float32),
            pltpu.VMEM((5, RS_SIZES[0], D), jnp.float32),
            pltpu.VMEM((5, B, SQ, H), jnp.float32),
            pltpu.SemaphoreType.DMA((10,)),
            pltpu.SemaphoreType.DMA((10,)),
            pltpu.SemaphoreType.DMA((5,)),
            pltpu.SemaphoreType.DMA((5,)),
        ],
    )

    return pl.pallas_call(
        body,
        grid_spec=grid_spec,
        out_shape=jax.ShapeDtypeStruct((B, SQ, H, D), jnp.float32),
        compiler_params=pltpu.CompilerParams(collective_id=0),
    )(rep, Q, K, V)


# device time: 45510 ns/iter; 2.0798x vs baseline; 1.1603x over previous
import jax
import jax.numpy as jnp
from jax import lax
from jax.experimental import pallas as pl
from jax.experimental.pallas import tpu as pltpu

B, SQ, H, D = 4, 32, 8, 128
ROWS = B * SQ * H
SROWS = ROWS // 2
SCALE = D ** -0.5
CHUNK = 512
MESHID = pl.DeviceIdType.MESH
S_SIZES = (256, 128, 64, 32, 16)


def kernel(Q, K, V):
    rep = (lax.axis_index("x") * 4 + lax.axis_index("y")).astype(jnp.int32)
    rep = rep.reshape((1,))

    def body(rep_ref, q_ref, k_ref, v_ref, out_ref,
             acc_o, acc_l, o_tmp, rbuf, rl, so_s, so_r, sl_s, sl_r):
        b = pl.program_id(0)
        my_x = lax.axis_index("x")
        my_y = lax.axis_index("y")
        my_z = lax.axis_index("z")
        partners = [
            (my_x ^ 1, my_y, my_z),
            (my_x, my_y ^ 1, my_z),
            (my_x, my_y, my_z ^ 1),
            (my_x, my_y ^ 2, my_z),
            (my_x, my_y, my_z ^ 2),
        ]
        bits = [
            my_x & 1,
            my_y & 1,
            my_z & 1,
            (my_y // 2) & 1,
            (my_z // 2) & 1,
        ]

        def rs_copy(s, r, cur):
            hr = S_SIZES[r]
            br = bits[r]
            send_off = s * SROWS + cur + (1 - br) * hr
            keep = cur + br * hr
            c = pltpu.make_async_remote_copy(
                src_ref=acc_o.at[pl.ds(send_off, hr)],
                dst_ref=rbuf.at[r, s, pl.ds(0, hr)],
                send_sem=so_s.at[2 * r + s], recv_sem=so_r.at[2 * r + s],
                device_id=partners[r], device_id_type=MESHID)
            return c, keep

        def ag_copy(s, i, cur):
            d = 4 - i
            gr = S_SIZES[4] * (2 ** i)
            r = 5 + i
            c = pltpu.make_async_remote_copy(
                src_ref=acc_o.at[pl.ds(s * SROWS + cur, gr)],
                dst_ref=acc_o.at[pl.ds(s * SROWS + cur, gr)],
                send_sem=so_s.at[2 * r + s], recv_sem=so_r.at[2 * r + s],
                device_id=partners[d], device_id_type=MESHID)
            return c, cur - bits[d] * gr, d

        @pl.when(b == 0)
        def _():
            bar = pltpu.get_barrier_semaphore()
            for p in partners:
                pl.semaphore_signal(bar, inc=1, device_id=p,
                                    device_id_type=MESHID)
            pl.semaphore_wait(bar, len(partners))

        for h in range(H):
            q = q_ref[0, :, h, :]
            k = k_ref[0, :, h, :]
            v = v_ref[0, :, h, :]
            s = lax.dot_general(q, k, (((1,), (1,)), ((), ())),
                                preferred_element_type=jnp.float32) * SCALE
            p = jnp.exp(s)
            acc_l[b, :, h] = jnp.sum(p, axis=1)
            o_tmp[:, h, :] = lax.dot_general(
                p, v, (((1,), (0,)), ((), ())),
                preferred_element_type=jnp.float32)
        acc_o[pl.ds(b * SQ * H, SQ * H)] = o_tmp[...].reshape(SQ * H, D)

        @pl.when(b == 1)
        def _():
            cA, _ = rs_copy(0, 0, 0)
            cA.start()

        @pl.when(b == B - 1)
        def _():
            dA, keepA = rs_copy(0, 0, 0)
            dB, keepB = rs_copy(1, 0, 0)
            dB.start()
            dL = pltpu.make_async_remote_copy(
                src_ref=acc_l, dst_ref=rl.at[0],
                send_sem=sl_s.at[0], recv_sem=sl_r.at[0],
                device_id=partners[0], device_id_type=MESHID)
            dL.start()
            curA = curB = 0
            for r in range(10):
                dA.wait()
                if r < 5:
                    hr = S_SIZES[r]
                    acc_o[pl.ds(0 * SROWS + keepA, hr)] += rbuf[r, 0, 0:hr]
                    curA = keepA
                else:
                    curA = nextA
                if r < 4:
                    dA, keepA = rs_copy(0, r + 1, curA)
                    dA.start()
                elif r < 9:
                    dA, nextA, _ = ag_copy(0, r + 1 - 5, curA)
                    dA.start()
                dB.wait()
                if r < 5:
                    hr = S_SIZES[r]
                    acc_o[pl.ds(1 * SROWS + keepB, hr)] += rbuf[r, 1, 0:hr]
                    curB = keepB
                else:
                    curB = nextB
                if r < 4:
                    dB, keepB = rs_copy(1, r + 1, curB)
                    dB.start()
                elif r < 9:
                    dB, nextB, _ = ag_copy(1, r + 1 - 5, curB)
                    dB.start()
                if r < 5:
                    dL.wait()
                    acc_l[...] += rl[r]
                    if r < 4:
                        dL = pltpu.make_async_remote_copy(
                            src_ref=acc_l, dst_ref=rl.at[r + 1],
                            send_sem=sl_s.at[r + 1], recv_sem=sl_r.at[r + 1],
                            device_id=partners[r + 1], device_id_type=MESHID)
                        dL.start()
            num = acc_o[...].reshape(B, SQ, H, D)
            out_ref[...] = num / acc_l[...][..., None]

    grid_spec = pltpu.PrefetchScalarGridSpec(
        num_scalar_prefetch=1,
        grid=(B,),
        in_specs=[
            pl.BlockSpec((1, SQ, H, D), lambda b, rep_ref: (b, 0, 0, 0)),
            pl.BlockSpec((1, CHUNK, H, D),
                         lambda b, rep_ref: (b, rep_ref[0], 0, 0)),
            pl.BlockSpec((1, CHUNK, H, D),
                         lambda b, rep_ref: (b, rep_ref[0], 0, 0)),
        ],
        out_specs=pl.BlockSpec((B, SQ, H, D), lambda b, rep_ref: (0, 0, 0, 0)),
        scratch_shapes=[
            pltpu.VMEM((ROWS, D), jnp.float32),
            pltpu.VMEM((B, SQ, H), jnp.float32),
            pltpu.VMEM((SQ, H, D), jnp.float32),
            pltpu.VMEM((5, 2, S_SIZES[0], D), jnp.float32),
            pltpu.VMEM((5, B, SQ, H), jnp.float32),
            pltpu.SemaphoreType.DMA((20,)),
            pltpu.SemaphoreType.DMA((20,)),
            pltpu.SemaphoreType.DMA((5,)),
            pltpu.SemaphoreType.DMA((5,)),
        ],
    )

    return pl.pallas_call(
        body,
        grid_spec=grid_spec,
        out_shape=jax.ShapeDtypeStruct((B, SQ, H, D), jnp.float32),
        compiler_params=pltpu.CompilerParams(collective_id=0),
    )(rep, Q, K, V)
